# baseline (speedup 1.0000x reference)
import jax
import jax.numpy as jnp
from jax import lax
from jax.experimental import pallas as pl
from jax.experimental.pallas import tpu as pltpu
from jax.experimental.pallas import tpu_sc as plsc

VOXEL_SIZE_Z = 0.04
MAX_OFFSET = 5.0

N_IMG = 8
C_FEAT = 128
H_IMG, W_IMG = 224, 384
HW = H_IMG * W_IMG
N_PTS = 25600
SENT = HW
PLANE_PAD = HW + 16

NC, NS = 2, 16
NW = NC * NS

_P1_CHUNK = 3200

_MESH = plsc.VectorSubcoreMesh(
    core_axis_name="c", subcore_axis_name="s", num_cores=NC, num_subcores=NS)


def _p1_body(d_hbm, lin_hbm, pk_hbm, v0_hbm, z_hbm, pts_hbm,
             fidx_hbm, valid_hbm, pts3_hbm,
             d_v, lin_v, pk_v, v0_v, z_v, px_v, py_v, pz_v,
             fidx_v, val_v, p3x_v, p3y_v, p3z_v):
  wid = lax.axis_index("s") * NC + lax.axis_index("c")
  for it in range(2):
    t = wid * 2 + it
    n = t // 8
    base = (t % 8) * _P1_CHUNK
    nbase = n * N_PTS + base
    pltpu.sync_copy(d_hbm.at[pl.ds(n * HW, HW)], d_v)
    pltpu.sync_copy(lin_hbm.at[pl.ds(nbase, _P1_CHUNK)], lin_v)
    pltpu.sync_copy(pk_hbm.at[pl.ds(nbase, _P1_CHUNK)], pk_v)
    pltpu.sync_copy(v0_hbm.at[pl.ds(nbase, _P1_CHUNK)], v0_v)
    pltpu.sync_copy(z_hbm.at[pl.ds(nbase, _P1_CHUNK)], z_v)
    pltpu.sync_copy(pts_hbm.at[pl.ds(base, _P1_CHUNK)], px_v)
    pltpu.sync_copy(pts_hbm.at[pl.ds(N_PTS + base, _P1_CHUNK)], py_v)
    pltpu.sync_copy(pts_hbm.at[pl.ds(2 * N_PTS + base, _P1_CHUNK)], pz_v)

    def body(k, _):
      sl = pl.ds(k * 16, 16)
      lin16 = lin_v[sl]
      dg = plsc.load_gather(d_v, [lin16])
      z16 = z_v[sl]
      cond = ((z16 > dg - jnp.float32(VOXEL_SIZE_Z))
              & (z16 < dg + jnp.float32(VOXEL_SIZE_Z)))
      v = cond & (v0_v[sl] != 0)
      fidx_v[sl] = jnp.where(v, lin_v[sl], SENT)
      val_v[sl] = v.astype(jnp.int32)
      vf = v.astype(jnp.float32)
      p3x_v[sl] = px_v[sl] * vf
      p3y_v[sl] = py_v[sl] * vf
      p3z_v[sl] = pz_v[sl] * vf
      return 0

    lax.fori_loop(0, _P1_CHUNK // 16, body, 0)
    pltpu.sync_copy(fidx_v, fidx_hbm.at[pl.ds(nbase, _P1_CHUNK)])
    pltpu.sync_copy(val_v, valid_hbm.at[pl.ds(nbase, _P1_CHUNK)])
    pltpu.sync_copy(p3x_v, pts3_hbm.at[pl.ds(n * 3 * N_PTS + base, _P1_CHUNK)])
    pltpu.sync_copy(p3y_v, pts3_hbm.at[pl.ds((n * 3 + 1) * N_PTS + base, _P1_CHUNK)])
    pltpu.sync_copy(p3z_v, pts3_hbm.at[pl.ds((n * 3 + 2) * N_PTS + base, _P1_CHUNK)])


_p1 = pl.kernel(
    _p1_body,
    out_type=(
        jax.ShapeDtypeStruct((N_IMG * N_PTS,), jnp.int32),
        jax.ShapeDtypeStruct((N_IMG * N_PTS,), jnp.int32),
        jax.ShapeDtypeStruct((N_IMG * 3 * N_PTS,), jnp.float32),
    ),
    mesh=_MESH,
    scratch_types=[
        pltpu.VMEM((HW,), jnp.float32),
        pltpu.VMEM((_P1_CHUNK,), jnp.int32),
        pltpu.VMEM((_P1_CHUNK,), jnp.int32),
        pltpu.VMEM((_P1_CHUNK,), jnp.int32),
        pltpu.VMEM((_P1_CHUNK,), jnp.float32),
        pltpu.VMEM((_P1_CHUNK,), jnp.float32),
        pltpu.VMEM((_P1_CHUNK,), jnp.float32),
        pltpu.VMEM((_P1_CHUNK,), jnp.float32),
        pltpu.VMEM((_P1_CHUNK,), jnp.int32),
        pltpu.VMEM((_P1_CHUNK,), jnp.int32),
        pltpu.VMEM((_P1_CHUNK,), jnp.float32),
        pltpu.VMEM((_P1_CHUNK,), jnp.float32),
        pltpu.VMEM((_P1_CHUNK,), jnp.float32),
    ],
    compiler_params=pltpu.CompilerParams(needs_layout_passes=False),
    name="backproject_mask_sc",
)


_OUT_CHUNK = 6400


def _p2_body(feat_hbm, fidx_hbm, vol_hbm,
             plane_v, fidx_v, out0_v, out1_v, sem0, sem1):
  wid = lax.axis_index("s") * NC + lax.axis_index("c")
  n = wid // 4
  c0 = (wid % 4) * 32
  pltpu.sync_copy(fidx_hbm.at[pl.ds(n * N_PTS, N_PTS)], fidx_v)
  plane_v[pl.ds(HW, 16)] = jnp.zeros((16,), jnp.float32)

  def plane_body(j, _):
    c = c0 + j
    nc = n * C_FEAT + c
    pltpu.sync_copy(feat_hbm.at[pl.ds(nc * HW, HW)], plane_v.at[pl.ds(0, HW)])
    outs = (out0_v, out1_v, out0_v, out1_v)
    sems = (sem0, sem1, sem0, sem1)
    cps = []
    for q in range(4):
      ob = outs[q]
      if q >= 2:
        cps[q - 2].wait()

      def gbody(k, _, q=q, ob=ob):
        for uu in range(4):
          off = k * 64 + uu * 16
          idx16 = fidx_v[pl.ds(q * _OUT_CHUNK + off, 16)]
          ob[pl.ds(off, 16)] = plsc.load_gather(plane_v, [idx16])
        return 0

      lax.fori_loop(0, _OUT_CHUNK // 64, gbody, 0)
      cps.append(pltpu.async_copy(
          ob, vol_hbm.at[pl.ds(nc * N_PTS + q * _OUT_CHUNK, _OUT_CHUNK)],
          sems[q]))
    cps[2].wait()
    cps[3].wait()
    return 0

  lax.fori_loop(0, 32, plane_body, 0)


_p2 = pl.kernel(
    _p2_body,
    out_type=jax.ShapeDtypeStruct((N_IMG * C_FEAT * N_PTS,), jnp.float32),
    mesh=_MESH,
    scratch_types=[
        pltpu.VMEM((PLANE_PAD,), jnp.float32),
        pltpu.VMEM((N_PTS,), jnp.int32),
        pltpu.VMEM((_OUT_CHUNK,), jnp.float32),
        pltpu.VMEM((_OUT_CHUNK,), jnp.float32),
        pltpu.SemaphoreType.DMA,
        pltpu.SemaphoreType.DMA,
    ],
    compiler_params=pltpu.CompilerParams(needs_layout_passes=False),
    name="backproject_gather_sc",
)


def kernel(features, points, projection, depth, offsets):
  n, C, H, W = features.shape
  nx, ny, nz = points.shape[-3:]
  off = jnp.tanh(offsets) * MAX_OFFSET
  off = jnp.broadcast_to(off, (n, off.shape[1], 2))
  pts = points.reshape(1, 3, -1)
  N = pts.shape[-1]
  ptsb = jnp.broadcast_to(pts, (n, 3, N))
  pts_h = jnp.concatenate([ptsb, jnp.ones((n, 1, N), dtype=ptsb.dtype)], axis=1)
  p23 = jnp.einsum('bij,bjn->bin', projection, pts_h)
  x = p23[:, 0] / p23[:, 2]
  y = p23[:, 1] / p23[:, 2]
  z = p23[:, 2]
  xi = jnp.round(x + off[:, :, 0]).astype(jnp.int32)
  yi = jnp.round(y + off[:, :, 1]).astype(jnp.int32)
  valid0 = (xi >= 0) & (yi >= 0) & (xi < W) & (yi < H) & (z > 0)
  d = jax.image.resize(depth[:, None, :, :], (n, 1, H, W), method='bilinear')[:, 0]
  xc = jnp.clip(xi, 0, W - 1)
  yc = jnp.clip(yi, 0, H - 1)
  lin = yc * W + xc
  pk = yc * 512 + xc

  fidx, valid_i, pts3 = _p1(
      d.reshape(-1), lin.reshape(-1), pk.reshape(-1),
      valid0.astype(jnp.int32).reshape(-1), z.reshape(-1), pts.reshape(-1))
  vol = _p2(features.reshape(-1), fidx)

  volume = vol.reshape(n, C, nx, ny, nz)
  valid_r = (valid_i != 0).reshape(n, 1, nx, ny, nz)
  pts3_r = pts3.reshape(n, 3, nx, ny, nz)
  return volume, valid_r, pts3_r


# speedup vs baseline: 1.0834x; 1.0834x over previous
import jax
import jax.numpy as jnp
from jax import lax
from jax.experimental import pallas as pl
from jax.experimental.pallas import tpu as pltpu
from jax.experimental.pallas import tpu_sc as plsc

VOXEL_SIZE_Z = 0.04
MAX_OFFSET = 5.0

N_IMG = 8
C_FEAT = 128
H_IMG, W_IMG = 224, 384
HW = H_IMG * W_IMG
N_PTS = 25600
SENT_PK = H_IMG * 512

NC, NS = 2, 16
NW = NC * NS

_P1_CHUNK = 3200

_MESH = plsc.VectorSubcoreMesh(
    core_axis_name="c", subcore_axis_name="s", num_cores=NC, num_subcores=NS)


def _p1_body(d_hbm, lin_hbm, pk_hbm, v0_hbm, z_hbm, pts_hbm,
             fidx_hbm, valid_hbm, pts3_hbm,
             d_v, lin_v, pk_v, v0_v, z_v, px_v, py_v, pz_v,
             fidx_v, val_v, p3x_v, p3y_v, p3z_v):
  wid = lax.axis_index("s") * NC + lax.axis_index("c")
  for it in range(2):
    t = wid * 2 + it
    n = t // 8
    base = (t % 8) * _P1_CHUNK
    nbase = n * N_PTS + base
    pltpu.sync_copy(d_hbm.at[pl.ds(n * HW, HW)], d_v)
    pltpu.sync_copy(lin_hbm.at[pl.ds(nbase, _P1_CHUNK)], lin_v)
    pltpu.sync_copy(pk_hbm.at[pl.ds(nbase, _P1_CHUNK)], pk_v)
    pltpu.sync_copy(v0_hbm.at[pl.ds(nbase, _P1_CHUNK)], v0_v)
    pltpu.sync_copy(z_hbm.at[pl.ds(nbase, _P1_CHUNK)], z_v)
    pltpu.sync_copy(pts_hbm.at[pl.ds(base, _P1_CHUNK)], px_v)
    pltpu.sync_copy(pts_hbm.at[pl.ds(N_PTS + base, _P1_CHUNK)], py_v)
    pltpu.sync_copy(pts_hbm.at[pl.ds(2 * N_PTS + base, _P1_CHUNK)], pz_v)

    def body(k, _):
      sl = pl.ds(k * 16, 16)
      lin16 = lin_v[sl]
      dg = plsc.load_gather(d_v, [lin16])
      z16 = z_v[sl]
      cond = ((z16 > dg - jnp.float32(VOXEL_SIZE_Z))
              & (z16 < dg + jnp.float32(VOXEL_SIZE_Z)))
      v = cond & (v0_v[sl] != 0)
      fidx_v[sl] = jnp.where(v, pk_v[sl], SENT_PK)
      val_v[sl] = v.astype(jnp.int32)
      vf = v.astype(jnp.float32)
      p3x_v[sl] = px_v[sl] * vf
      p3y_v[sl] = py_v[sl] * vf
      p3z_v[sl] = pz_v[sl] * vf
      return 0

    lax.fori_loop(0, _P1_CHUNK // 16, body, 0)
    pltpu.sync_copy(fidx_v, fidx_hbm.at[pl.ds(nbase, _P1_CHUNK)])
    pltpu.sync_copy(val_v, valid_hbm.at[pl.ds(nbase, _P1_CHUNK)])
    pltpu.sync_copy(p3x_v, pts3_hbm.at[pl.ds(n * 3 * N_PTS + base, _P1_CHUNK)])
    pltpu.sync_copy(p3y_v, pts3_hbm.at[pl.ds((n * 3 + 1) * N_PTS + base, _P1_CHUNK)])
    pltpu.sync_copy(p3z_v, pts3_hbm.at[pl.ds((n * 3 + 2) * N_PTS + base, _P1_CHUNK)])


_p1 = pl.kernel(
    _p1_body,
    out_type=(
        jax.ShapeDtypeStruct((N_IMG * N_PTS,), jnp.int32),
        jax.ShapeDtypeStruct((N_IMG * N_PTS,), jnp.int32),
        jax.ShapeDtypeStruct((N_IMG * 3 * N_PTS,), jnp.float32),
    ),
    mesh=_MESH,
    scratch_types=[
        pltpu.VMEM((HW,), jnp.float32),
        pltpu.VMEM((_P1_CHUNK,), jnp.int32),
        pltpu.VMEM((_P1_CHUNK,), jnp.int32),
        pltpu.VMEM((_P1_CHUNK,), jnp.int32),
        pltpu.VMEM((_P1_CHUNK,), jnp.float32),
        pltpu.VMEM((_P1_CHUNK,), jnp.float32),
        pltpu.VMEM((_P1_CHUNK,), jnp.float32),
        pltpu.VMEM((_P1_CHUNK,), jnp.float32),
        pltpu.VMEM((_P1_CHUNK,), jnp.int32),
        pltpu.VMEM((_P1_CHUNK,), jnp.int32),
        pltpu.VMEM((_P1_CHUNK,), jnp.float32),
        pltpu.VMEM((_P1_CHUNK,), jnp.float32),
        pltpu.VMEM((_P1_CHUNK,), jnp.float32),
    ],
    compiler_params=pltpu.CompilerParams(needs_layout_passes=False),
    name="backproject_mask_sc",
)


_OUT_CHUNK = 6400


def _p2_body(feat_hbm, fidx_hbm, vol_hbm,
             plane_v, fidx_v, out0_v, out1_v, sem0, sem1):
  wid = lax.axis_index("s") * NC + lax.axis_index("c")
  n = wid // 4
  c0 = (wid % 4) * 32
  pltpu.sync_copy(fidx_hbm.at[pl.ds(n * N_PTS, N_PTS)], fidx_v)
  plane_v[H_IMG, pl.ds(0, 16)] = jnp.zeros((16,), jnp.float32)

  def plane_body(j, _):
    c = c0 + j
    nc = n * C_FEAT + c
    pltpu.sync_copy(feat_hbm.at[n, c], plane_v.at[pl.ds(0, H_IMG), :])
    outs = (out0_v, out1_v, out0_v, out1_v)
    sems = (sem0, sem1, sem0, sem1)
    cps = []
    for q in range(4):
      ob = outs[q]
      if q >= 2:
        cps[q - 2].wait()

      def gbody(k, _, q=q, ob=ob):
        for uu in range(4):
          off = k * 64 + uu * 16
          pk16 = fidx_v[pl.ds(q * _OUT_CHUNK + off, 16)]
          y16 = pk16 >> 9
          x16 = pk16 & 511
          ob[pl.ds(off, 16)] = plsc.load_gather(plane_v, [y16, x16])
        return 0

      lax.fori_loop(0, _OUT_CHUNK // 64, gbody, 0)
      cps.append(pltpu.async_copy(
          ob, vol_hbm.at[pl.ds(nc * N_PTS + q * _OUT_CHUNK, _OUT_CHUNK)],
          sems[q]))
    cps[2].wait()
    cps[3].wait()
    return 0

  lax.fori_loop(0, 32, plane_body, 0)


_p2 = pl.kernel(
    _p2_body,
    out_type=jax.ShapeDtypeStruct((N_IMG * C_FEAT * N_PTS,), jnp.float32),
    mesh=_MESH,
    scratch_types=[
        pltpu.VMEM((H_IMG + 1, W_IMG), jnp.float32),
        pltpu.VMEM((N_PTS,), jnp.int32),
        pltpu.VMEM((_OUT_CHUNK,), jnp.float32),
        pltpu.VMEM((_OUT_CHUNK,), jnp.float32),
        pltpu.SemaphoreType.DMA,
        pltpu.SemaphoreType.DMA,
    ],
    compiler_params=pltpu.CompilerParams(needs_layout_passes=False),
    name="backproject_gather_sc",
)


def kernel(features, points, projection, depth, offsets):
  n, C, H, W = features.shape
  nx, ny, nz = points.shape[-3:]
  off = jnp.tanh(offsets) * MAX_OFFSET
  off = jnp.broadcast_to(off, (n, off.shape[1], 2))
  pts = points.reshape(1, 3, -1)
  N = pts.shape[-1]
  ptsb = jnp.broadcast_to(pts, (n, 3, N))
  pts_h = jnp.concatenate([ptsb, jnp.ones((n, 1, N), dtype=ptsb.dtype)], axis=1)
  p23 = jnp.einsum('bij,bjn->bin', projection, pts_h)
  x = p23[:, 0] / p23[:, 2]
  y = p23[:, 1] / p23[:, 2]
  z = p23[:, 2]
  xi = jnp.round(x + off[:, :, 0]).astype(jnp.int32)
  yi = jnp.round(y + off[:, :, 1]).astype(jnp.int32)
  valid0 = (xi >= 0) & (yi >= 0) & (xi < W) & (yi < H) & (z > 0)
  d = jax.image.resize(depth[:, None, :, :], (n, 1, H, W), method='bilinear')[:, 0]
  xc = jnp.clip(xi, 0, W - 1)
  yc = jnp.clip(yi, 0, H - 1)
  lin = yc * W + xc
  pk = yc * 512 + xc

  fidx, valid_i, pts3 = _p1(
      d.reshape(-1), lin.reshape(-1), pk.reshape(-1),
      valid0.astype(jnp.int32).reshape(-1), z.reshape(-1), pts.reshape(-1))
  vol = _p2(features, fidx)
  volume = vol.reshape(n, C, nx, ny, nz)

  valid_r = (valid_i != 0).reshape(n, 1, nx, ny, nz)
  pts3_r = pts3.reshape(n, 3, nx, ny, nz)
  return volume, valid_r, pts3_r


# speedup vs baseline: 1.0891x; 1.0053x over previous
import jax
import jax.numpy as jnp
from jax import lax
from jax.experimental import pallas as pl
from jax.experimental.pallas import tpu as pltpu
from jax.experimental.pallas import tpu_sc as plsc

VOXEL_SIZE_Z = 0.04
MAX_OFFSET = 5.0

N_IMG = 8
C_FEAT = 128
H_IMG, W_IMG = 224, 384
HW = H_IMG * W_IMG
N_PTS = 25600
SENT_PK = H_IMG * 512

NC, NS = 2, 16
NW = NC * NS

_P1_CHUNK = 3200

_MESH = plsc.VectorSubcoreMesh(
    core_axis_name="c", subcore_axis_name="s", num_cores=NC, num_subcores=NS)


def _p1_body(d_hbm, lin_hbm, pk_hbm, v0_hbm, z_hbm, pts_hbm,
             fidx_hbm, valid_hbm, pts3_hbm,
             d_v, lin_v, pk_v, v0_v, z_v, px_v, py_v, pz_v,
             fidx_v, val_v, p3x_v, p3y_v, p3z_v):
  wid = lax.axis_index("s") * NC + lax.axis_index("c")
  for it in range(2):
    t = wid * 2 + it
    n = t // 8
    base = (t % 8) * _P1_CHUNK
    nbase = n * N_PTS + base
    pltpu.sync_copy(d_hbm.at[pl.ds(n * HW, HW)], d_v)
    pltpu.sync_copy(lin_hbm.at[pl.ds(nbase, _P1_CHUNK)], lin_v)
    pltpu.sync_copy(pk_hbm.at[pl.ds(nbase, _P1_CHUNK)], pk_v)
    pltpu.sync_copy(v0_hbm.at[pl.ds(nbase, _P1_CHUNK)], v0_v)
    pltpu.sync_copy(z_hbm.at[pl.ds(nbase, _P1_CHUNK)], z_v)
    pltpu.sync_copy(pts_hbm.at[pl.ds(base, _P1_CHUNK)], px_v)
    pltpu.sync_copy(pts_hbm.at[pl.ds(N_PTS + base, _P1_CHUNK)], py_v)
    pltpu.sync_copy(pts_hbm.at[pl.ds(2 * N_PTS + base, _P1_CHUNK)], pz_v)

    def body(k, _):
      sl = pl.ds(k * 16, 16)
      lin16 = lin_v[sl]
      dg = plsc.load_gather(d_v, [lin16])
      z16 = z_v[sl]
      cond = ((z16 > dg - jnp.float32(VOXEL_SIZE_Z))
              & (z16 < dg + jnp.float32(VOXEL_SIZE_Z)))
      v = cond & (v0_v[sl] != 0)
      fidx_v[sl] = jnp.where(v, pk_v[sl], SENT_PK)
      val_v[sl] = v.astype(jnp.int32)
      vf = v.astype(jnp.float32)
      p3x_v[sl] = px_v[sl] * vf
      p3y_v[sl] = py_v[sl] * vf
      p3z_v[sl] = pz_v[sl] * vf
      return 0

    lax.fori_loop(0, _P1_CHUNK // 16, body, 0)
    pltpu.sync_copy(fidx_v, fidx_hbm.at[pl.ds(nbase, _P1_CHUNK)])
    pltpu.sync_copy(val_v, valid_hbm.at[pl.ds(nbase, _P1_CHUNK)])
    pltpu.sync_copy(p3x_v, pts3_hbm.at[pl.ds(n * 3 * N_PTS + base, _P1_CHUNK)])
    pltpu.sync_copy(p3y_v, pts3_hbm.at[pl.ds((n * 3 + 1) * N_PTS + base, _P1_CHUNK)])
    pltpu.sync_copy(p3z_v, pts3_hbm.at[pl.ds((n * 3 + 2) * N_PTS + base, _P1_CHUNK)])


_p1 = pl.kernel(
    _p1_body,
    out_type=(
        jax.ShapeDtypeStruct((N_IMG * N_PTS,), jnp.int32),
        jax.ShapeDtypeStruct((N_IMG * N_PTS,), jnp.int32),
        jax.ShapeDtypeStruct((N_IMG * 3 * N_PTS,), jnp.float32),
    ),
    mesh=_MESH,
    scratch_types=[
        pltpu.VMEM((HW,), jnp.float32),
        pltpu.VMEM((_P1_CHUNK,), jnp.int32),
        pltpu.VMEM((_P1_CHUNK,), jnp.int32),
        pltpu.VMEM((_P1_CHUNK,), jnp.int32),
        pltpu.VMEM((_P1_CHUNK,), jnp.float32),
        pltpu.VMEM((_P1_CHUNK,), jnp.float32),
        pltpu.VMEM((_P1_CHUNK,), jnp.float32),
        pltpu.VMEM((_P1_CHUNK,), jnp.float32),
        pltpu.VMEM((_P1_CHUNK,), jnp.int32),
        pltpu.VMEM((_P1_CHUNK,), jnp.int32),
        pltpu.VMEM((_P1_CHUNK,), jnp.float32),
        pltpu.VMEM((_P1_CHUNK,), jnp.float32),
        pltpu.VMEM((_P1_CHUNK,), jnp.float32),
    ],
    compiler_params=pltpu.CompilerParams(needs_layout_passes=False),
    name="backproject_mask_sc",
)


_OUT_CHUNK = 6400


def _p2_body(feat_hbm, fidx_hbm, vol_hbm,
             plane_v, fidx_v, out0_v, out1_v, sem0, sem1):
  wid = lax.axis_index("s") * NC + lax.axis_index("c")
  n = wid // 4
  c0 = (wid % 4) * 32
  pltpu.sync_copy(fidx_hbm.at[pl.ds(n * N_PTS, N_PTS)], fidx_v)
  plane_v[H_IMG, pl.ds(0, 16)] = jnp.zeros((16,), jnp.float32)

  def plane_body(j, _):
    c = c0 + j
    nc = n * C_FEAT + c
    pltpu.sync_copy(feat_hbm.at[n, c], plane_v.at[pl.ds(0, H_IMG), :])

    @pl.when(j > 0)
    def _():
      pltpu.make_async_copy(
          vol_hbm.at[pl.ds(0, _OUT_CHUNK)], out0_v, sem0).wait()
      pltpu.make_async_copy(
          vol_hbm.at[pl.ds(0, _OUT_CHUNK)], out1_v, sem1).wait()

    outs = (out0_v, out1_v, out0_v, out1_v)
    sems = (sem0, sem1, sem0, sem1)
    cps = []
    for q in range(4):
      ob = outs[q]
      if q >= 2:
        cps[q - 2].wait()

      def gbody(k, _, q=q, ob=ob):
        for uu in range(4):
          off = k * 64 + uu * 16
          pk16 = fidx_v[pl.ds(q * _OUT_CHUNK + off, 16)]
          y16 = pk16 >> 9
          x16 = pk16 & 511
          ob[pl.ds(off, 16)] = plsc.load_gather(plane_v, [y16, x16])
        return 0

      lax.fori_loop(0, _OUT_CHUNK // 64, gbody, 0)
      cps.append(pltpu.async_copy(
          ob, vol_hbm.at[pl.ds(nc * N_PTS + q * _OUT_CHUNK, _OUT_CHUNK)],
          sems[q]))
    return 0

  lax.fori_loop(0, 32, plane_body, 0)
  pltpu.make_async_copy(vol_hbm.at[pl.ds(0, _OUT_CHUNK)], out0_v, sem0).wait()
  pltpu.make_async_copy(vol_hbm.at[pl.ds(0, _OUT_CHUNK)], out1_v, sem1).wait()


_p2 = pl.kernel(
    _p2_body,
    out_type=jax.ShapeDtypeStruct((N_IMG * C_FEAT * N_PTS,), jnp.float32),
    mesh=_MESH,
    scratch_types=[
        pltpu.VMEM((H_IMG + 1, W_IMG), jnp.float32),
        pltpu.VMEM((N_PTS,), jnp.int32),
        pltpu.VMEM((_OUT_CHUNK,), jnp.float32),
        pltpu.VMEM((_OUT_CHUNK,), jnp.float32),
        pltpu.SemaphoreType.DMA,
        pltpu.SemaphoreType.DMA,
    ],
    compiler_params=pltpu.CompilerParams(needs_layout_passes=False),
    name="backproject_gather_sc",
)


def kernel(features, points, projection, depth, offsets):
  n, C, H, W = features.shape
  nx, ny, nz = points.shape[-3:]
  off = jnp.tanh(offsets) * MAX_OFFSET
  off = jnp.broadcast_to(off, (n, off.shape[1], 2))
  pts = points.reshape(1, 3, -1)
  N = pts.shape[-1]
  ptsb = jnp.broadcast_to(pts, (n, 3, N))
  pts_h = jnp.concatenate([ptsb, jnp.ones((n, 1, N), dtype=ptsb.dtype)], axis=1)
  p23 = jnp.einsum('bij,bjn->bin', projection, pts_h)
  x = p23[:, 0] / p23[:, 2]
  y = p23[:, 1] / p23[:, 2]
  z = p23[:, 2]
  xi = jnp.round(x + off[:, :, 0]).astype(jnp.int32)
  yi = jnp.round(y + off[:, :, 1]).astype(jnp.int32)
  valid0 = (xi >= 0) & (yi >= 0) & (xi < W) & (yi < H) & (z > 0)
  d = jax.image.resize(depth[:, None, :, :], (n, 1, H, W), method='bilinear')[:, 0]
  xc = jnp.clip(xi, 0, W - 1)
  yc = jnp.clip(yi, 0, H - 1)
  lin = yc * W + xc
  pk = yc * 512 + xc

  fidx, valid_i, pts3 = _p1(
      d.reshape(-1), lin.reshape(-1), pk.reshape(-1),
      valid0.astype(jnp.int32).reshape(-1), z.reshape(-1), pts.reshape(-1))
  vol = _p2(features, fidx)
  volume = vol.reshape(n, C, nx, ny, nz)

  valid_r = (valid_i != 0).reshape(n, 1, nx, ny, nz)
  pts3_r = pts3.reshape(n, 3, nx, ny, nz)
  return volume, valid_r, pts3_r
